# baseline (speedup 1.0000x reference)
import functools
import numpy as np
import jax
import jax.numpy as jnp
from jax import lax
from jax.experimental import pallas as pl
from jax.experimental.pallas import tpu as pltpu
from jax.experimental.pallas import tpu_sc as plsc

_MAX_DIST = 18
_BOARD = 19
_N = _BOARD * _BOARD
_PAD = 384
_NW = 32
_ROWS_PER_W = _PAD // _NW
_STRIPE = _ROWS_PER_W * _PAD
_L = 16


def _build_scatter_lists():
    dirs = [(-1, 0), (1, 0), (0, -1), (0, 1),
            (-1, -1), (-1, 1), (1, -1), (1, 1)]
    rr, cc = np.meshgrid(np.arange(_BOARD), np.arange(_BOARD), indexing="ij")
    p_flat = rr * _BOARD + cc
    per_w = [[] for _ in range(_NW)]
    for d, (dr, dc) in enumerate(dirs):
        koff = 0 if d < 4 else _MAX_DIST
        for t in range(1, _MAX_DIST + 1):
            tr = rr + dr * t
            tc = cc + dc * t
            valid = (tr >= 0) & (tr < _BOARD) & (tc >= 0) & (tc < _BOARD)
            k = koff + t - 1
            for p, q in zip(p_flat[valid], (tr * _BOARD + tc)[valid]):
                w = q // _ROWS_PER_W
                off = (q - w * _ROWS_PER_W) * _PAD + p
                per_w[w].append(off | (k << 16))
    lmax = max(len(e) for e in per_w)
    lmax = -(-lmax // _L) * _L
    packed = np.zeros((_NW, lmax), np.int32)
    for w, ent in enumerate(per_w):
        pad = [(_STRIPE + (j % _L)) | (36 << 16)
               for j in range(lmax - len(ent))]
        packed[w] = np.asarray(ent + pad, np.int32)
    return packed


_PACKED_NP = _build_scatter_lists()
_LMAX = _PACKED_NP.shape[1]
_SC_MESH = plsc.VectorSubcoreMesh(core_axis_name="c", subcore_axis_name="s")


@functools.partial(
    pl.kernel,
    mesh=_SC_MESH,
    out_type=jax.ShapeDtypeStruct((_PAD * _PAD,), jnp.float32),
    scratch_types=[
        pltpu.VMEM((_LMAX,), jnp.int32),
        pltpu.VMEM((64,), jnp.float32),
        pltpu.VMEM((_STRIPE + _L,), jnp.float32),
    ],
    compiler_params=pltpu.CompilerParams(needs_layout_passes=False),
)
def _sc_build_mt(w_hbm, packed_hbm, out_hbm, pk_v, w_v, buf_v):
    wid = lax.axis_index("s") * 2 + lax.axis_index("c")
    pltpu.sync_copy(w_hbm, w_v)
    pltpu.sync_copy(packed_hbm.at[wid], pk_v)

    def _zero(i, _):
        buf_v[pl.ds(i * _L, _L)] = jnp.zeros((_L,), jnp.float32)
        return _

    lax.fori_loop(0, (_STRIPE + _L) // _L, _zero, 0)

    def _scatter(i, _):
        pk = pk_v[pl.ds(i * _L, _L)]
        off = lax.bitwise_and(pk, 0xFFFF)
        kk = lax.shift_right_logical(pk, 16)
        vals = plsc.load_gather(w_v, [kk])
        plsc.store_scatter(buf_v, [off], vals)
        return _

    lax.fori_loop(0, _LMAX // _L, _scatter, 0)
    pltpu.sync_copy(buf_v.at[pl.ds(0, _STRIPE)],
                    out_hbm.at[pl.ds(wid * _STRIPE, _STRIPE)])


def _mm_body(mt_ref, x_ref, out_ref):
    mtv = mt_ref[0:_N, 0:_N]
    out_ref[...] = jnp.dot(x_ref[...], mtv, preferred_element_type=jnp.float32)


def kernel(x, weight):
    B, C, H, W = x.shape
    xf = x.reshape(B * C, H * W)
    wexp = jnp.concatenate(
        [weight[0], weight[1], jnp.zeros((64 - 2 * _MAX_DIST,), weight.dtype)])
    mt_flat = _sc_build_mt(wexp, jnp.asarray(_PACKED_NP))
    mt = mt_flat.reshape(_PAD, _PAD)
    out = pl.pallas_call(
        _mm_body,
        out_shape=jax.ShapeDtypeStruct((B * C, H * W), jnp.float32),
        in_specs=[
            pl.BlockSpec(memory_space=pltpu.VMEM),
            pl.BlockSpec(memory_space=pltpu.VMEM),
        ],
        out_specs=pl.BlockSpec(memory_space=pltpu.VMEM),
    )(mt, xf)
    return out.reshape(B, C, H, W)


# speedup vs baseline: 2.6227x; 2.6227x over previous
import numpy as np
import jax
import jax.numpy as jnp
from jax.experimental import pallas as pl
from jax.experimental.pallas import tpu as pltpu

_MAX_DIST = 18
_BOARD = 19
_N = _BOARD * _BOARD


def _build_code_map():
    dirs = [(-1, 0), (1, 0), (0, -1), (0, 1),
            (-1, -1), (-1, 1), (1, -1), (1, 1)]
    tm = np.zeros((_N, _N), np.int8)
    rr, cc = np.meshgrid(np.arange(_BOARD), np.arange(_BOARD), indexing="ij")
    p_flat = rr * _BOARD + cc
    for d, (dr, dc) in enumerate(dirs):
        off = 0 if d < 4 else _MAX_DIST
        for t in range(1, _MAX_DIST + 1):
            tr = rr + dr * t
            tc = cc + dc * t
            valid = (tr >= 0) & (tr < _BOARD) & (tc >= 0) & (tc < _BOARD)
            p = p_flat[valid]
            q = (tr * _BOARD + tc)[valid]
            tm[q, p] = off + t
    return tm


_TM_NP = _build_code_map()


def _body(w_ref, tm_ref, x_ref, out_ref):
    tm = tm_ref[...].astype(jnp.bfloat16)
    mt = jnp.zeros((_N, _N), jnp.bfloat16)
    for t in range(1, _MAX_DIST + 1):
        mt = jnp.where(tm == t, w_ref[0, t - 1].astype(jnp.bfloat16), mt)
        mt = jnp.where(tm == _MAX_DIST + t,
                       w_ref[1, t - 1].astype(jnp.bfloat16), mt)
    out_ref[...] = jnp.dot(x_ref[...], mt.astype(jnp.float32),
                           preferred_element_type=jnp.float32)


def kernel(x, weight):
    B, C, H, W = x.shape
    xf = x.reshape(B * C, H * W)
    out = pl.pallas_call(
        _body,
        out_shape=jax.ShapeDtypeStruct((B * C, H * W), jnp.float32),
        in_specs=[
            pl.BlockSpec(memory_space=pltpu.SMEM),
            pl.BlockSpec(memory_space=pltpu.VMEM),
            pl.BlockSpec(memory_space=pltpu.VMEM),
        ],
        out_specs=pl.BlockSpec(memory_space=pltpu.VMEM),
    )(weight, jnp.asarray(_TM_NP), xf)
    return out.reshape(B, C, H, W)
